# baseline (speedup 1.0000x reference)
import functools

import jax
import jax.numpy as jnp
from jax import lax
from jax.experimental import pallas as pl
from jax.experimental.pallas import tpu as pltpu
from jax.experimental.pallas import tpu_sc as plsc

N, C, H, W = 8, 19, 512, 512
NPIX = N * H * W
TOTAL = C * NPIX
NBINS = 16384
CODES = 4 * NBINS
NTILES = 32
W_SPAN = TOTAL // NTILES
TOTALW = TOTAL // 2
W_SPANW = TOTALW // NTILES
CHUNK = 16384
NCHUNKS = W_SPANW // CHUNK
HB = 64

assert W_SPAN * NTILES == TOTAL and NCHUNKS * CHUNK == W_SPANW


def _code_body(x_ref, t_ref, o_ref):
    x = x_ref[0]
    t = t_ref[0]
    m = jnp.max(x, axis=0, keepdims=True)
    e = jnp.exp(x - m)
    p = e / jnp.sum(e, axis=0, keepdims=True)
    ci = lax.broadcasted_iota(jnp.int32, (C, HB, W), 0)
    fg = t[None, :, :] == ci
    err = jnp.where(fg, 1.0 - p, p)
    b = jnp.clip(jnp.floor(err * float(NBINS)).astype(jnp.int32), 0, NBINS - 1)
    code = b + jnp.where(fg, NBINS, 0) + (ci & 1) * (2 * NBINS)
    o_ref[:, 0] = code.astype(jnp.uint16)


def _compute_codes(inp, tgt):
    return pl.pallas_call(
        _code_body,
        grid=(N, H // HB),
        in_specs=[
            pl.BlockSpec((1, C, HB, W), lambda n, h: (n, 0, h, 0)),
            pl.BlockSpec((1, HB, W), lambda n, h: (n, h, 0)),
        ],
        out_specs=pl.BlockSpec((C, 1, HB, W), lambda n, h: (0, n, h, 0)),
        out_shape=jax.ShapeDtypeStruct((C, N, H, W), jnp.uint16),
        compiler_params=pltpu.CompilerParams(
            dimension_semantics=("parallel", "parallel")),
    )(inp, tgt)


def _hist_body(codes_hbm, zeros_hbm, out_hbm, buf0, buf1, hist, sem0, sem1):
    cid = lax.axis_index("c")
    sid = lax.axis_index("s")
    wid = sid * 2 + cid
    base = wid * W_SPANW
    pltpu.sync_copy(zeros_hbm, hist)
    ones = jnp.full((16,), 1, jnp.int32)
    bufs = (buf0, buf1)
    sems = (sem0, sem1)

    pltpu.async_copy(codes_hbm.at[pl.ds(base, CHUNK)], buf0, sem0)
    pltpu.async_copy(
        codes_hbm.at[pl.ds(pl.multiple_of(base + CHUNK, CHUNK), CHUNK)], buf1, sem1)

    def outer(k2, carry):
        for b in range(2):
            k = k2 * 2 + b
            buf, sem = bufs[b], sems[b]
            pltpu.make_async_copy(codes_hbm.at[pl.ds(0, CHUNK)], buf, sem).wait()

            def inner(i, c2, buf=buf):
                w = buf[pl.ds(i * 16, 16)]
                plsc.addupdate_scatter(hist, [w & 0xFFFF], ones)
                plsc.addupdate_scatter(hist, [lax.shift_right_logical(w, 16)], ones)
                return c2

            carry = lax.fori_loop(0, CHUNK // 16, inner, carry, unroll=8)

            @pl.when(k + 2 < NCHUNKS)
            def _(buf=buf, sem=sem, k=k):
                off = pl.multiple_of(base + (k + 2) * CHUNK, CHUNK)
                pltpu.async_copy(codes_hbm.at[pl.ds(off, CHUNK)], buf, sem)

        return carry

    lax.fori_loop(0, NCHUNKS // 2, outer, jnp.int32(0))
    pltpu.sync_copy(hist, out_hbm.at[wid])


def _histograms(codes_flat, zeros):
    mesh = plsc.VectorSubcoreMesh(core_axis_name="c", subcore_axis_name="s")
    k = functools.partial(
        pl.kernel,
        mesh=mesh,
        out_type=jax.ShapeDtypeStruct((NTILES, CODES), jnp.int32),
        scratch_types=[
            pltpu.VMEM((CHUNK,), jnp.int32),
            pltpu.VMEM((CHUNK,), jnp.int32),
            pltpu.VMEM((CODES,), jnp.int32),
            pltpu.SemaphoreType.DMA,
            pltpu.SemaphoreType.DMA,
        ],
        compiler_params=pltpu.CompilerParams(needs_layout_passes=False),
    )(_hist_body)
    return k(codes_flat, zeros)


def _loss_body(h_ref, o_ref):
    r = lax.broadcasted_iota(jnp.int32, (128, 128), 0)
    col = lax.broadcasted_iota(jnp.int32, (128, 128), 1)
    upper = (r <= col).astype(jnp.float32)
    strict_lower = (col < r).astype(jnp.float32)
    center = ((r * 128 + col).astype(jnp.float32) + 0.5) * (1.0 / NBINS)

    def dot(a, b):
        return lax.dot_general(a, b, (((1,), (0,)), ((), ())),
                               preferred_element_type=jnp.float32)

    def dot_int_l(a_i32, b01):
        acc = dot(jnp.float32(0x10000) * ((a_i32 >> 16) & 0xFF).astype(jnp.float32), b01)
        acc += dot(jnp.float32(0x100) * ((a_i32 >> 8) & 0xFF).astype(jnp.float32), b01)
        return acc + dot((a_i32 & 0xFF).astype(jnp.float32), b01)

    def dot_int_r(a01, b_i32):
        acc = dot(a01, jnp.float32(0x10000) * ((b_i32 >> 16) & 0xFF).astype(jnp.float32))
        acc += dot(a01, jnp.float32(0x100) * ((b_i32 >> 8) & 0xFF).astype(jnp.float32))
        return acc + dot(a01, (b_i32 & 0xFF).astype(jnp.float32))

    def prefix(a_i32):
        rowp = dot_int_l(a_i32, upper)
        rowtot = rowp[:, 127:128].astype(jnp.int32)
        rowoff = dot_int_r(strict_lower, rowtot)
        return rowp + rowoff

    num = jnp.float32(0.0)
    den = jnp.float32(0.0)
    for c in range(C):
        lo, hi = c * NPIX, (c + 1) * NPIX
        tiles = [t for t in range(NTILES)
                 if t * W_SPAN < hi and (t + 1) * W_SPAN > lo]
        q0 = (c % 2) * 2
        f_i = h_ref[tiles[0], q0 + 1]
        m_i = h_ref[tiles[0], q0]
        for t in tiles[1:]:
            f_i = f_i + h_ref[t, q0 + 1]
            m_i = m_i + h_ref[t, q0]
        mt_i = m_i + f_i
        f = f_i.astype(jnp.float32)
        m = mt_i.astype(jnp.float32)
        g = jnp.sum(f)
        tot_m = jnp.sum(m)
        m_incl = prefix(mt_i)
        f_incl = prefix(f_i)
        k_in = tot_m - m_incl + m
        f_in = g - f_incl + f
        k_out = k_in - m
        f_out = f_in - f

        def jac(k, fgc):
            d = jnp.maximum(g + k - fgc, 1.0)
            return jnp.where(k > 0, 1.0 - (g - fgc) / d, 0.0)

        loss_c = jnp.sum(center * (jac(k_in, f_in) - jac(k_out, f_out)))
        pres = (g > 0).astype(jnp.float32)
        num = num + loss_c * pres
        den = den + pres
    o_ref[...] = jnp.full((1, 1), num / jnp.maximum(den, 1.0), jnp.float32)


def _combine(hists):
    return pl.pallas_call(
        _loss_body,
        out_shape=jax.ShapeDtypeStruct((1, 1), jnp.float32),
    )(hists)


def kernel(input, target):
    codes = _compute_codes(input, target.astype(jnp.int32))
    packed = lax.bitcast_convert_type(codes.reshape(TOTALW, 2), jnp.int32)
    zeros = jnp.zeros((CODES,), jnp.int32)
    hists = _histograms(packed, zeros)
    loss = _combine(hists.reshape(NTILES, 4, 128, 128))
    return loss[0, 0]


# speedup vs baseline: 26.0780x; 26.0780x over previous
import functools

import jax
import jax.numpy as jnp
from jax import lax
from jax.experimental import pallas as pl
from jax.experimental.pallas import tpu as pltpu
from jax.experimental.pallas import tpu_sc as plsc

N, C, H, W = 8, 19, 512, 512
NPIX = N * H * W
TOTAL = C * NPIX
NBINS = 16384
CODES = 4 * NBINS
NTILES = 32
W_SPAN = TOTAL // NTILES
TOTALW = TOTAL // 2
W_SPANW = TOTALW // NTILES
CHUNK = 16384
NCHUNKS = W_SPANW // CHUNK
HB = 64

assert W_SPAN * NTILES == TOTAL and NCHUNKS * CHUNK == W_SPANW


def _code_body(x_ref, t_ref, o_ref):
    x = x_ref[0]
    t = t_ref[0]
    m = jnp.max(x, axis=0, keepdims=True)
    e = jnp.exp(x - m)
    p = e / jnp.sum(e, axis=0, keepdims=True)
    ci = lax.broadcasted_iota(jnp.int32, (C, HB, W), 0)
    fg = t[None, :, :] == ci
    err = jnp.where(fg, 1.0 - p, p)
    b = jnp.clip(jnp.floor(err * float(NBINS)).astype(jnp.int32), 0, NBINS - 1)
    code = b + jnp.where(fg, NBINS, 0) + (ci & 1) * (2 * NBINS)
    o_ref[:, 0] = code[:, :HB // 2, :] | (code[:, HB // 2:, :] << 16)


def _compute_codes(inp, tgt):
    return pl.pallas_call(
        _code_body,
        grid=(N, H // HB),
        in_specs=[
            pl.BlockSpec((1, C, HB, W), lambda n, h: (n, 0, h, 0)),
            pl.BlockSpec((1, HB, W), lambda n, h: (n, h, 0)),
        ],
        out_specs=pl.BlockSpec((C, 1, HB // 2, W), lambda n, h: (0, n, h, 0)),
        out_shape=jax.ShapeDtypeStruct((C, N, H // 2, W), jnp.int32),
        compiler_params=pltpu.CompilerParams(
            dimension_semantics=("parallel", "parallel")),
    )(inp, tgt)


def _hist_body(codes_hbm, zeros_hbm, out_hbm, buf0, buf1, hist, sem0, sem1):
    cid = lax.axis_index("c")
    sid = lax.axis_index("s")
    wid = sid * 2 + cid
    base = wid * W_SPANW
    pltpu.sync_copy(zeros_hbm, hist)
    ones = jnp.full((16,), 1, jnp.int32)
    bufs = (buf0, buf1)
    sems = (sem0, sem1)

    pltpu.async_copy(codes_hbm.at[pl.ds(base, CHUNK)], buf0, sem0)
    pltpu.async_copy(
        codes_hbm.at[pl.ds(pl.multiple_of(base + CHUNK, CHUNK), CHUNK)], buf1, sem1)

    def outer(k2, carry):
        for b in range(2):
            k = k2 * 2 + b
            buf, sem = bufs[b], sems[b]
            pltpu.make_async_copy(codes_hbm.at[pl.ds(0, CHUNK)], buf, sem).wait()

            def inner(i, c2, buf=buf):
                w = buf[pl.ds(i * 16, 16)]
                plsc.addupdate_scatter(hist, [w & 0xFFFF], ones)
                plsc.addupdate_scatter(hist, [lax.shift_right_logical(w, 16)], ones)
                return c2

            carry = lax.fori_loop(0, CHUNK // 16, inner, carry, unroll=8)

            @pl.when(k + 2 < NCHUNKS)
            def _(buf=buf, sem=sem, k=k):
                off = pl.multiple_of(base + (k + 2) * CHUNK, CHUNK)
                pltpu.async_copy(codes_hbm.at[pl.ds(off, CHUNK)], buf, sem)

        return carry

    lax.fori_loop(0, NCHUNKS // 2, outer, jnp.int32(0))
    pltpu.sync_copy(hist, out_hbm.at[wid])


def _histograms(codes_flat, zeros):
    mesh = plsc.VectorSubcoreMesh(core_axis_name="c", subcore_axis_name="s")
    k = functools.partial(
        pl.kernel,
        mesh=mesh,
        out_type=jax.ShapeDtypeStruct((NTILES, CODES), jnp.int32),
        scratch_types=[
            pltpu.VMEM((CHUNK,), jnp.int32),
            pltpu.VMEM((CHUNK,), jnp.int32),
            pltpu.VMEM((CODES,), jnp.int32),
            pltpu.SemaphoreType.DMA,
            pltpu.SemaphoreType.DMA,
        ],
        compiler_params=pltpu.CompilerParams(needs_layout_passes=False),
    )(_hist_body)
    return k(codes_flat, zeros)


def _loss_body(h_ref, o_ref):
    r = lax.broadcasted_iota(jnp.int32, (128, 128), 0)
    col = lax.broadcasted_iota(jnp.int32, (128, 128), 1)
    upper = (r <= col).astype(jnp.float32)
    strict_lower = (col < r).astype(jnp.float32)
    center = ((r * 128 + col).astype(jnp.float32) + 0.5) * (1.0 / NBINS)

    def dot(a, b):
        return lax.dot_general(a, b, (((1,), (0,)), ((), ())),
                               preferred_element_type=jnp.float32)

    def dot_int_l(a_i32, b01):
        acc = dot(jnp.float32(0x10000) * ((a_i32 >> 16) & 0xFF).astype(jnp.float32), b01)
        acc += dot(jnp.float32(0x100) * ((a_i32 >> 8) & 0xFF).astype(jnp.float32), b01)
        return acc + dot((a_i32 & 0xFF).astype(jnp.float32), b01)

    def dot_int_r(a01, b_i32):
        acc = dot(a01, jnp.float32(0x10000) * ((b_i32 >> 16) & 0xFF).astype(jnp.float32))
        acc += dot(a01, jnp.float32(0x100) * ((b_i32 >> 8) & 0xFF).astype(jnp.float32))
        return acc + dot(a01, (b_i32 & 0xFF).astype(jnp.float32))

    def prefix(a_i32):
        rowp = dot_int_l(a_i32, upper)
        rowtot = rowp[:, 127:128].astype(jnp.int32)
        rowoff = dot_int_r(strict_lower, rowtot)
        return rowp + rowoff

    num = jnp.float32(0.0)
    den = jnp.float32(0.0)
    for c in range(C):
        lo, hi = c * NPIX, (c + 1) * NPIX
        tiles = [t for t in range(NTILES)
                 if t * W_SPAN < hi and (t + 1) * W_SPAN > lo]
        q0 = (c % 2) * 2
        f_i = h_ref[tiles[0], q0 + 1]
        m_i = h_ref[tiles[0], q0]
        for t in tiles[1:]:
            f_i = f_i + h_ref[t, q0 + 1]
            m_i = m_i + h_ref[t, q0]
        mt_i = m_i + f_i
        f = f_i.astype(jnp.float32)
        m = mt_i.astype(jnp.float32)
        g = jnp.sum(f)
        tot_m = jnp.sum(m)
        m_incl = prefix(mt_i)
        f_incl = prefix(f_i)
        k_in = tot_m - m_incl + m
        f_in = g - f_incl + f
        k_out = k_in - m
        f_out = f_in - f

        def jac(k, fgc):
            d = jnp.maximum(g + k - fgc, 1.0)
            return jnp.where(k > 0, 1.0 - (g - fgc) / d, 0.0)

        loss_c = jnp.sum(center * (jac(k_in, f_in) - jac(k_out, f_out)))
        pres = (g > 0).astype(jnp.float32)
        num = num + loss_c * pres
        den = den + pres
    o_ref[...] = jnp.full((1, 1), num / jnp.maximum(den, 1.0), jnp.float32)


def _combine(hists):
    return pl.pallas_call(
        _loss_body,
        out_shape=jax.ShapeDtypeStruct((1, 1), jnp.float32),
    )(hists)


def kernel(input, target):
    packed = _compute_codes(input, target.astype(jnp.int32))
    zeros = jnp.zeros((CODES,), jnp.int32)
    hists = _histograms(packed.reshape(-1), zeros)
    loss = _combine(hists.reshape(NTILES, 4, 128, 128))
    return loss[0, 0]
